# baseline (speedup 1.0000x reference)
import functools

import jax
import jax.numpy as jnp
from jax import lax
from jax.experimental import pallas as pl
from jax.experimental.pallas import tpu as pltpu
from jax.experimental.pallas import tpu_sc as plsc

_NUM_ENT = 1000000
_NUM_REL = 1000
_D = 64
_B = 16384

_NW = 32
_BPW = _B // _NW
_CHUNK = 128
_NCH = _BPW // _CHUNK
_G = _BPW // 16


def _transe_body(hs_hbm, rs_hbm, ts_hbm, ent_hbm, rel_hbm, out_hbm,
                 hs_v, rs_v, ts_v, h_v, r_v, t_v, o_v, sem):
    wid = lax.axis_index("s") * 2 + lax.axis_index("c")
    base = wid * _BPW

    for c in range(_NCH):
        off = base + c * _CHUNK
        pltpu.sync_copy(hs_hbm.at[pl.ds(off, _CHUNK)], hs_v.at[c])
        pltpu.sync_copy(rs_hbm.at[pl.ds(off, _CHUNK)], rs_v.at[c])
        pltpu.sync_copy(ts_hbm.at[pl.ds(off, _CHUNK)], ts_v.at[c])

    copies = []
    for c in range(_NCH):
        dst = pl.ds(c * _CHUNK, _CHUNK)
        copies.append(pltpu.async_copy(ent_hbm.at[hs_v.at[c]], h_v.at[dst], sem))
        copies.append(pltpu.async_copy(rel_hbm.at[rs_v.at[c]], r_v.at[dst], sem))
        copies.append(pltpu.async_copy(ent_hbm.at[ts_v.at[c]], t_v.at[dst], sem))
    for cp in copies:
        cp.wait()

    iota16 = lax.iota(jnp.int32, 16)
    half = jnp.float32(0.5)
    threehalf = jnp.float32(1.5)
    magic = jnp.int32(0x5F3759DF)

    def group_body(g, carry):
        rows = g * 16 + iota16

        def j_body(j, acc):
            col = jnp.full((16,), j, jnp.int32)
            hv = plsc.load_gather(h_v, [rows, col])
            rv = plsc.load_gather(r_v, [rows, col])
            tv = plsc.load_gather(t_v, [rows, col])
            d = (hv + rv) - tv
            return acc + d * d

        acc = lax.fori_loop(0, _D, j_body, jnp.zeros((16,), jnp.float32))

        bits = lax.bitcast_convert_type(acc, jnp.int32)
        y = lax.bitcast_convert_type(magic - (bits >> 1), jnp.float32)
        hx = half * acc
        for _ in range(3):
            y = y * (threehalf - hx * y * y)
        o_v[pl.ds(g * 16, 16)] = acc * y
        return carry

    lax.fori_loop(0, _G, group_body, jnp.int32(0))
    pltpu.sync_copy(o_v, out_hbm.at[pl.ds(base, _BPW)])


@jax.jit
def _transe_call(hs, rs, ts, ent_embs, rel_embs):
    mesh = plsc.VectorSubcoreMesh(core_axis_name="c", subcore_axis_name="s")
    fn = functools.partial(
        pl.kernel,
        mesh=mesh,
        out_type=jax.ShapeDtypeStruct((_B,), jnp.float32),
        compiler_params=pltpu.CompilerParams(
            use_tc_tiling_on_sc=False, needs_layout_passes=False
        ),
        scratch_types=[
            pltpu.VMEM((_NCH, _CHUNK), jnp.int32),
            pltpu.VMEM((_NCH, _CHUNK), jnp.int32),
            pltpu.VMEM((_NCH, _CHUNK), jnp.int32),
            pltpu.VMEM((_BPW, _D), jnp.float32),
            pltpu.VMEM((_BPW, _D), jnp.float32),
            pltpu.VMEM((_BPW, _D), jnp.float32),
            pltpu.VMEM((_BPW,), jnp.float32),
            pltpu.SemaphoreType.DMA,
        ],
    )(_transe_body)
    return fn(hs, rs, ts, ent_embs, rel_embs)


def kernel(hs, rs, ts, ent_embs, rel_embs):
    out = _transe_call(hs, rs, ts, ent_embs, rel_embs)
    return out.reshape(-1, 1)


# speedup vs baseline: 1.6181x; 1.6181x over previous
import functools

import jax
import jax.numpy as jnp
from jax import lax
from jax.experimental import pallas as pl
from jax.experimental.pallas import tpu as pltpu
from jax.experimental.pallas import tpu_sc as plsc

_NUM_ENT = 1000000
_NUM_REL = 1000
_D = 64
_B = 16384

_NW = 32
_BPW = _B // _NW
_G = _BPW // 16


def _transe_body(hs_hbm, rs_hbm, ts_hbm, ent_hbm, rel_hbm, out_hbm,
                 hs_v, rs_v, ts_v, hb, rb, tb, o_v, sem0, sem1):
    wid = lax.axis_index("s") * 2 + lax.axis_index("c")
    base = wid * _BPW

    pltpu.sync_copy(hs_hbm.at[pl.ds(base, _BPW)], hs_v)
    pltpu.sync_copy(rs_hbm.at[pl.ds(base, _BPW)], rs_v)
    pltpu.sync_copy(ts_hbm.at[pl.ds(base, _BPW)], ts_v)

    sems = (sem0, sem1)

    def issue(p, g):
        sem = sems[p]
        hv = hs_v[pl.ds(g * 16, 16)]
        rv = rs_v[pl.ds(g * 16, 16)]
        tv = ts_v[pl.ds(g * 16, 16)]
        for k in range(16):
            pltpu.async_copy(ent_hbm.at[hv[k]], hb.at[p, k, pl.ds(0, _D)], sem)
            pltpu.async_copy(rel_hbm.at[rv[k]], rb.at[p, k, pl.ds(0, _D)], sem)
            pltpu.async_copy(ent_hbm.at[tv[k]], tb.at[p, k, pl.ds(0, _D)], sem)

    def drain(p):
        sem = sems[p]
        for k in range(16):
            for buf, src in ((hb, ent_hbm), (rb, rel_hbm), (tb, ent_hbm)):
                pltpu.make_async_copy(
                    src.at[0], buf.at[p, k, pl.ds(0, _D)], sem
                ).wait()

    iota16 = lax.iota(jnp.int32, 16)
    half = jnp.float32(0.5)
    threehalf = jnp.float32(1.5)
    magic = jnp.int32(0x5F3759DF)

    def compute(p, g):
        pvec = jnp.full((16,), p, jnp.int32)

        def j_body(j, acc):
            col = jnp.full((16,), j, jnp.int32)
            hvv = plsc.load_gather(hb, [pvec, iota16, col])
            rvv = plsc.load_gather(rb, [pvec, iota16, col])
            tvv = plsc.load_gather(tb, [pvec, iota16, col])
            d = (hvv + rvv) - tvv
            return acc + d * d

        acc = lax.fori_loop(0, _D, j_body, jnp.zeros((16,), jnp.float32))

        bits = lax.bitcast_convert_type(acc, jnp.int32)
        y = lax.bitcast_convert_type(magic - (bits >> 1), jnp.float32)
        hx = half * acc
        for _ in range(3):
            y = y * (threehalf - hx * y * y)
        o_v[pl.ds(g * 16, 16)] = acc * y

    issue(0, 0)

    def pair_body(i, carry):
        g0 = 2 * i
        issue(1, g0 + 1)
        drain(0)
        compute(0, g0)

        @pl.when(i < _G // 2 - 1)
        def _():
            issue(0, g0 + 2)

        drain(1)
        compute(1, g0 + 1)
        return carry

    lax.fori_loop(0, _G // 2, pair_body, jnp.int32(0))
    pltpu.sync_copy(o_v, out_hbm.at[pl.ds(base, _BPW)])


@jax.jit
def _transe_call(hs, rs, ts, ent_embs, rel_embs):
    mesh = plsc.VectorSubcoreMesh(core_axis_name="c", subcore_axis_name="s")
    fn = functools.partial(
        pl.kernel,
        mesh=mesh,
        out_type=jax.ShapeDtypeStruct((_B,), jnp.float32),
        compiler_params=pltpu.CompilerParams(needs_layout_passes=False),
        scratch_types=[
            pltpu.VMEM((_BPW,), jnp.int32),
            pltpu.VMEM((_BPW,), jnp.int32),
            pltpu.VMEM((_BPW,), jnp.int32),
            pltpu.VMEM((2, 16, 128), jnp.float32),
            pltpu.VMEM((2, 16, 128), jnp.float32),
            pltpu.VMEM((2, 16, 128), jnp.float32),
            pltpu.VMEM((_BPW,), jnp.float32),
            pltpu.SemaphoreType.DMA,
            pltpu.SemaphoreType.DMA,
        ],
    )(_transe_body)
    return fn(hs, rs, ts, ent_embs, rel_embs)


def kernel(hs, rs, ts, ent_embs, rel_embs):
    out = _transe_call(hs, rs, ts, ent_embs, rel_embs)
    return out.reshape(-1, 1)
